# Initial kernel scaffold; baseline (speedup 1.0000x reference)
import jax
import jax.numpy as jnp
from jax.experimental import pallas as pl


def kernel(x, edge_index, W1, b1, W2, b2):
    raise NotImplementedError("write your pallas kernel here")


# speedup vs baseline: 13.0382x; 13.0382x over previous
import functools

import jax
import jax.numpy as jnp
from jax import lax
from jax.experimental import pallas as pl
from jax.experimental.pallas import tpu as pltpu
from jax.experimental.pallas import tpu_sc as plsc

N_NODES = 10000
N_EDGES = 320000
D = 128

NC, NS = 2, 16
NW = NC * NS
E_PER_W = N_EDGES // NW
K = 80
NB = E_PER_W // K
RPT = N_NODES // NS

@functools.cache
def _sc_kernels():
    mesh = plsc.VectorSubcoreMesh(core_axis_name="c", subcore_axis_name="s",
                                  num_cores=NC, num_subcores=NS)
    sc_params = pltpu.CompilerParams(use_tc_tiling_on_sc=False)

    @functools.partial(
        pl.kernel,
        out_type=jax.ShapeDtypeStruct((NC, N_NODES, 16), jnp.float32),
        mesh=mesh,
        scratch_types=[
            pltpu.VMEM_SHARED((N_NODES, 16), jnp.float32),
            pltpu.VMEM((K,), jnp.int32),
            pltpu.VMEM((K, 16), jnp.float32),
        ],
        compiler_params=sc_params,
    )
    def sc_degree(dst_hbm, ones_hbm, zeros_hbm, out_hbm, acc, didx, ones_v):
        c = lax.axis_index("c")
        s = lax.axis_index("s")
        wid = c * NS + s
        row0 = s * RPT
        pltpu.sync_copy(zeros_hbm, acc.at[pl.ds(row0, RPT)])
        pltpu.sync_copy(ones_hbm, ones_v)
        plsc.subcore_barrier()
        e_base = wid * E_PER_W

        @pl.loop(0, NB)
        def _(b):
            pltpu.sync_copy(dst_hbm.at[pl.ds(e_base + b * K, K)], didx)
            pltpu.sync_copy(ones_v, acc.at[didx], add=True)

        plsc.subcore_barrier()
        pltpu.sync_copy(acc.at[pl.ds(row0, RPT)],
                        out_hbm.at[c, pl.ds(row0, RPT)])

    @functools.partial(
        pl.kernel,
        out_type=jax.ShapeDtypeStruct((NC, N_NODES, D), jnp.float32),
        mesh=mesh,
        scratch_types=[
            pltpu.VMEM_SHARED((N_NODES, D), jnp.float32),
            pltpu.VMEM((K,), jnp.int32),
            pltpu.VMEM((K,), jnp.int32),
            pltpu.VMEM((K, D), jnp.float32),
            pltpu.SemaphoreType.DMA,
        ],
        compiler_params=sc_params,
    )
    def sc_aggregate(y_hbm, src_hbm, dst_hbm, zeros_hbm, out_hbm,
                     acc, sidx, didx, rows, sem):
        c = lax.axis_index("c")
        s = lax.axis_index("s")
        wid = c * NS + s
        row0 = s * RPT
        pltpu.sync_copy(zeros_hbm, acc.at[pl.ds(row0, RPT)])
        plsc.subcore_barrier()
        e_base = wid * E_PER_W

        @pl.loop(0, NB)
        def _(b):
            e0 = e_base + b * K
            pltpu.sync_copy(src_hbm.at[pl.ds(e0, K)], sidx)
            pltpu.sync_copy(dst_hbm.at[pl.ds(e0, K)], didx)
            pltpu.async_copy(y_hbm.at[sidx], rows, sem).wait()
            pltpu.sync_copy(rows, acc.at[didx], add=True)

        plsc.subcore_barrier()
        pltpu.sync_copy(acc.at[pl.ds(row0, RPT)],
                        out_hbm.at[c, pl.ds(row0, RPT)])

    return sc_degree, sc_aggregate


RB = 2000
GRID = N_NODES // RB


def _dis_block(degp):
    return lax.rsqrt(degp[0, :, 0] + degp[1, :, 0] + 1.0)


def _tc_first_body(degp_ref, x_ref, w_ref, y_ref):
    dis = _dis_block(degp_ref[...])
    y = jnp.dot(x_ref[...], w_ref[...], preferred_element_type=jnp.float32)
    y_ref[...] = y * dis[:, None]


def _tc_mid_body(degp_ref, agg_ref, y_ref, b_ref, w_ref, out_ref):
    dis = _dis_block(degp_ref[...])
    agg = agg_ref[0] + agg_ref[1] + y_ref[...]
    h = jnp.maximum(agg * dis[:, None] + b_ref[...], 0.0)
    out_ref[...] = jnp.dot(h, w_ref[...],
                           preferred_element_type=jnp.float32) * dis[:, None]


def _tc_last_body(degp_ref, agg_ref, y_ref, b_ref, out_ref):
    dis = _dis_block(degp_ref[...])
    agg = agg_ref[0] + agg_ref[1] + y_ref[...]
    out_ref[...] = jnp.maximum(agg * dis[:, None] + b_ref[...], 0.0)


_degp_spec = pl.BlockSpec((NC, RB, 16), lambda i: (0, i, 0))
_row_spec = pl.BlockSpec((RB, D), lambda i: (i, 0))
_agg_spec = pl.BlockSpec((NC, RB, D), lambda i: (0, i, 0))
_w_spec = pl.BlockSpec((D, D), lambda i: (0, 0))
_b_spec = pl.BlockSpec((1, D), lambda i: (0, 0))

_tc_first = pl.pallas_call(
    _tc_first_body,
    grid=(GRID,),
    in_specs=[_degp_spec, _row_spec, _w_spec],
    out_specs=_row_spec,
    out_shape=jax.ShapeDtypeStruct((N_NODES, D), jnp.float32),
)

_tc_mid = pl.pallas_call(
    _tc_mid_body,
    grid=(GRID,),
    in_specs=[_degp_spec, _agg_spec, _row_spec, _b_spec, _w_spec],
    out_specs=_row_spec,
    out_shape=jax.ShapeDtypeStruct((N_NODES, D), jnp.float32),
)

_tc_last = pl.pallas_call(
    _tc_last_body,
    grid=(GRID,),
    in_specs=[_degp_spec, _agg_spec, _row_spec, _b_spec],
    out_specs=_row_spec,
    out_shape=jax.ShapeDtypeStruct((N_NODES, D), jnp.float32),
)


def kernel(x, edge_index, W1, b1, W2, b2):
    src = edge_index[0].astype(jnp.int32)
    dst = edge_index[1].astype(jnp.int32)
    ones16 = jnp.ones((K, 16), jnp.float32)
    zeros16 = jnp.zeros((RPT, 16), jnp.float32)
    zerosD = jnp.zeros((RPT, D), jnp.float32)
    b1r = b1.reshape(1, D)
    b2r = b2.reshape(1, D)

    sc_degree, sc_aggregate = _sc_kernels()
    degp = sc_degree(dst, ones16, zeros16)
    y1 = _tc_first(degp, x, W1)
    agg1 = sc_aggregate(y1, src, dst, zerosD)
    y2 = _tc_mid(degp, agg1, y1, b1r, W2)
    agg2 = sc_aggregate(y2, src, dst, zerosD)
    out = _tc_last(degp, agg2, y2, b2r)
    return out


# speedup vs baseline: 34.2787x; 2.6291x over previous
import functools

import jax
import jax.numpy as jnp
from jax import lax
from jax.experimental import pallas as pl
from jax.experimental.pallas import tpu as pltpu
from jax.experimental.pallas import tpu_sc as plsc

N_NODES = 10000
N_EDGES = 320000
D = 128

NC, NS = 2, 16
NW = NC * NS
E_PER_W = N_EDGES // NW
K = 40
NB = E_PER_W // K
NBUF = 5
RPT = N_NODES // NS

@functools.cache
def _sc_kernels():
    mesh = plsc.VectorSubcoreMesh(core_axis_name="c", subcore_axis_name="s",
                                  num_cores=NC, num_subcores=NS)
    sc_params = pltpu.CompilerParams(use_tc_tiling_on_sc=False)

    @functools.partial(
        pl.kernel,
        out_type=jax.ShapeDtypeStruct((NC, N_NODES, 16), jnp.float32),
        mesh=mesh,
        scratch_types=[
            pltpu.VMEM_SHARED((N_NODES, 16), jnp.float32),
            pltpu.VMEM((NB, K), jnp.int32),
            pltpu.VMEM((K, 16), jnp.float32),
        ],
        compiler_params=sc_params,
    )
    def sc_degree(dst_hbm, ones_hbm, zeros_hbm, out_hbm, acc, didx, ones_v):
        c = lax.axis_index("c")
        s = lax.axis_index("s")
        wid = c * NS + s
        row0 = s * RPT
        pltpu.sync_copy(zeros_hbm, acc.at[pl.ds(row0, RPT)])
        pltpu.sync_copy(ones_hbm, ones_v)
        pltpu.sync_copy(dst_hbm.at[wid], didx)
        plsc.subcore_barrier()

        @pl.loop(0, NB)
        def _(b):
            pltpu.sync_copy(ones_v, acc.at[didx.at[b]], add=True)

        plsc.subcore_barrier()
        pltpu.sync_copy(acc.at[pl.ds(row0, RPT)],
                        out_hbm.at[c, pl.ds(row0, RPT)])

    @functools.partial(
        pl.kernel,
        out_type=jax.ShapeDtypeStruct((NC, N_NODES, D), jnp.float32),
        mesh=mesh,
        scratch_types=[
            pltpu.VMEM_SHARED((N_NODES, D), jnp.float32),
            pltpu.VMEM((NB, K), jnp.int32),
            pltpu.VMEM((NB, K), jnp.int32),
            pltpu.VMEM((NBUF, K, D), jnp.float32),
            [pltpu.SemaphoreType.DMA] * NBUF,
        ],
        compiler_params=sc_params,
    )
    def sc_aggregate(y_hbm, src_hbm, dst_hbm, zeros_hbm, out_hbm,
                     acc, sidx, didx, rows, sems):
        c = lax.axis_index("c")
        s = lax.axis_index("s")
        wid = c * NS + s
        row0 = s * RPT
        pltpu.sync_copy(zeros_hbm, acc.at[pl.ds(row0, RPT)])
        pltpu.sync_copy(src_hbm.at[wid], sidx)
        pltpu.sync_copy(dst_hbm.at[wid], didx)
        plsc.subcore_barrier()

        def fire(b, j):
            pltpu.async_copy(y_hbm.at[sidx.at[b]], rows.at[j], sems[j])

        def drain(b, j):
            pltpu.make_async_copy(y_hbm.at[sidx.at[b]], rows.at[j],
                                  sems[j]).wait()
            pltpu.sync_copy(rows.at[j], acc.at[didx.at[b]], add=True)

        for j in range(NBUF):
            fire(j, j)

        @pl.loop(0, NB // NBUF - 1)
        def _(g):
            b0 = g * NBUF
            for j in range(NBUF):
                drain(b0 + j, j)
                fire(b0 + j + NBUF, j)

        for j in range(NBUF):
            drain(NB - NBUF + j, j)

        plsc.subcore_barrier()
        pltpu.sync_copy(acc.at[pl.ds(row0, RPT)],
                        out_hbm.at[c, pl.ds(row0, RPT)])

    return sc_degree, sc_aggregate


RB = 2000
GRID = N_NODES // RB


def _dis_block(degp):
    return lax.rsqrt(degp[0, :, 0] + degp[1, :, 0] + 1.0)


def _tc_first_body(degp_ref, x_ref, w_ref, y_ref):
    dis = _dis_block(degp_ref[...])
    y = jnp.dot(x_ref[...], w_ref[...], preferred_element_type=jnp.float32)
    y_ref[...] = y * dis[:, None]


def _tc_mid_body(degp_ref, agg_ref, y_ref, b_ref, w_ref, out_ref):
    dis = _dis_block(degp_ref[...])
    agg = agg_ref[0] + agg_ref[1] + y_ref[...]
    h = jnp.maximum(agg * dis[:, None] + b_ref[...], 0.0)
    out_ref[...] = jnp.dot(h, w_ref[...],
                           preferred_element_type=jnp.float32) * dis[:, None]


def _tc_last_body(degp_ref, agg_ref, y_ref, b_ref, out_ref):
    dis = _dis_block(degp_ref[...])
    agg = agg_ref[0] + agg_ref[1] + y_ref[...]
    out_ref[...] = jnp.maximum(agg * dis[:, None] + b_ref[...], 0.0)


_degp_spec = pl.BlockSpec((NC, RB, 16), lambda i: (0, i, 0))
_row_spec = pl.BlockSpec((RB, D), lambda i: (i, 0))
_agg_spec = pl.BlockSpec((NC, RB, D), lambda i: (0, i, 0))
_w_spec = pl.BlockSpec((D, D), lambda i: (0, 0))
_b_spec = pl.BlockSpec((1, D), lambda i: (0, 0))

_tc_first = pl.pallas_call(
    _tc_first_body,
    grid=(GRID,),
    in_specs=[_degp_spec, _row_spec, _w_spec],
    out_specs=_row_spec,
    out_shape=jax.ShapeDtypeStruct((N_NODES, D), jnp.float32),
)

_tc_mid = pl.pallas_call(
    _tc_mid_body,
    grid=(GRID,),
    in_specs=[_degp_spec, _agg_spec, _row_spec, _b_spec, _w_spec],
    out_specs=_row_spec,
    out_shape=jax.ShapeDtypeStruct((N_NODES, D), jnp.float32),
)

_tc_last = pl.pallas_call(
    _tc_last_body,
    grid=(GRID,),
    in_specs=[_degp_spec, _agg_spec, _row_spec, _b_spec],
    out_specs=_row_spec,
    out_shape=jax.ShapeDtypeStruct((N_NODES, D), jnp.float32),
)


def kernel(x, edge_index, W1, b1, W2, b2):
    src = edge_index[0].astype(jnp.int32).reshape(NW, NB, K)
    dst = edge_index[1].astype(jnp.int32).reshape(NW, NB, K)
    ones16 = jnp.ones((K, 16), jnp.float32)
    zeros16 = jnp.zeros((RPT, 16), jnp.float32)
    zerosD = jnp.zeros((RPT, D), jnp.float32)
    b1r = b1.reshape(1, D)
    b2r = b2.reshape(1, D)

    sc_degree, sc_aggregate = _sc_kernels()
    degp = sc_degree(dst, ones16, zeros16)
    y1 = _tc_first(degp, x, W1)
    agg1 = sc_aggregate(y1, src, dst, zerosD)
    y2 = _tc_mid(degp, agg1, y1, b1r, W2)
    agg2 = sc_aggregate(y2, src, dst, zerosD)
    out = _tc_last(degp, agg2, y2, b2r)
    return out


# speedup vs baseline: 35.8622x; 1.0462x over previous
import functools

import jax
import jax.numpy as jnp
from jax import lax
from jax.experimental import pallas as pl
from jax.experimental.pallas import tpu as pltpu
from jax.experimental.pallas import tpu_sc as plsc

N_NODES = 10000
N_EDGES = 320000
D = 128

NC, NS = 2, 16
NW = NC * NS
E_PER_W = N_EDGES // NW
K = 40
NB = E_PER_W // K
NBUF = 5
KD = 125
NBD = E_PER_W // KD
RPT = N_NODES // NS

@functools.cache
def _sc_kernels():
    mesh = plsc.VectorSubcoreMesh(core_axis_name="c", subcore_axis_name="s",
                                  num_cores=NC, num_subcores=NS)
    sc_params = pltpu.CompilerParams(use_tc_tiling_on_sc=False)

    @functools.partial(
        pl.kernel,
        out_type=jax.ShapeDtypeStruct((NC, N_NODES, 16), jnp.float32),
        mesh=mesh,
        scratch_types=[
            pltpu.VMEM_SHARED((N_NODES, 16), jnp.float32),
            pltpu.VMEM((NBD, KD), jnp.int32),
            pltpu.VMEM((KD, 16), jnp.float32),
            pltpu.SemaphoreType.DMA,
        ],
        compiler_params=sc_params,
    )
    def sc_degree(dst_hbm, ones_hbm, zeros_hbm, out_hbm, acc, didx, ones_v,
                  sem):
        c = lax.axis_index("c")
        s = lax.axis_index("s")
        wid = c * NS + s
        row0 = s * RPT
        pltpu.sync_copy(zeros_hbm, acc.at[pl.ds(row0, RPT)])
        pltpu.sync_copy(ones_hbm, ones_v)
        pltpu.sync_copy(dst_hbm.at[wid], didx)
        plsc.subcore_barrier()

        @pl.loop(0, NBD)
        def _(b):
            pltpu.async_copy(ones_v, acc.at[didx.at[b]], sem, add=True)

        @pl.loop(0, NBD)
        def _(b):
            pltpu.make_async_copy(ones_v, acc.at[didx.at[b]], sem).wait()

        plsc.subcore_barrier()
        pltpu.sync_copy(acc.at[pl.ds(row0, RPT)],
                        out_hbm.at[c, pl.ds(row0, RPT)])

    @functools.partial(
        pl.kernel,
        out_type=jax.ShapeDtypeStruct((NC, N_NODES, D), jnp.float32),
        mesh=mesh,
        scratch_types=[
            pltpu.VMEM_SHARED((N_NODES, D), jnp.float32),
            pltpu.VMEM((NB, K), jnp.int32),
            pltpu.VMEM((NB, K), jnp.int32),
            pltpu.VMEM((NBUF, K, D), jnp.float32),
            [pltpu.SemaphoreType.DMA] * NBUF,
        ],
        compiler_params=sc_params,
    )
    def sc_aggregate(y_hbm, src_hbm, dst_hbm, zeros_hbm, out_hbm,
                     acc, sidx, didx, rows, sems):
        c = lax.axis_index("c")
        s = lax.axis_index("s")
        wid = c * NS + s
        row0 = s * RPT
        pltpu.sync_copy(zeros_hbm, acc.at[pl.ds(row0, RPT)])
        pltpu.sync_copy(src_hbm.at[wid], sidx)
        pltpu.sync_copy(dst_hbm.at[wid], didx)
        plsc.subcore_barrier()

        def fire(b, j):
            pltpu.async_copy(y_hbm.at[sidx.at[b]], rows.at[j], sems[j])

        def drain(b, j):
            pltpu.make_async_copy(y_hbm.at[sidx.at[b]], rows.at[j],
                                  sems[j]).wait()
            pltpu.sync_copy(rows.at[j], acc.at[didx.at[b]], add=True)

        for j in range(NBUF):
            fire(j, j)

        @pl.loop(0, NB // NBUF - 1)
        def _(g):
            b0 = g * NBUF
            for j in range(NBUF):
                drain(b0 + j, j)
                fire(b0 + j + NBUF, j)

        for j in range(NBUF):
            drain(NB - NBUF + j, j)

        plsc.subcore_barrier()
        pltpu.sync_copy(acc.at[pl.ds(row0, RPT)],
                        out_hbm.at[c, pl.ds(row0, RPT)])

    return sc_degree, sc_aggregate


RB = 2000
GRID = N_NODES // RB


def _dis_block(degp):
    return lax.rsqrt(degp[0, :, 0] + degp[1, :, 0] + 1.0)


def _tc_first_body(degp_ref, x_ref, w_ref, y_ref):
    dis = _dis_block(degp_ref[...])
    y = jnp.dot(x_ref[...], w_ref[...], preferred_element_type=jnp.float32)
    y_ref[...] = y * dis[:, None]


def _tc_mid_body(degp_ref, agg_ref, y_ref, b_ref, w_ref, out_ref):
    dis = _dis_block(degp_ref[...])
    agg = agg_ref[0] + agg_ref[1] + y_ref[...]
    h = jnp.maximum(agg * dis[:, None] + b_ref[...], 0.0)
    out_ref[...] = jnp.dot(h, w_ref[...],
                           preferred_element_type=jnp.float32) * dis[:, None]


def _tc_last_body(degp_ref, agg_ref, y_ref, b_ref, out_ref):
    dis = _dis_block(degp_ref[...])
    agg = agg_ref[0] + agg_ref[1] + y_ref[...]
    out_ref[...] = jnp.maximum(agg * dis[:, None] + b_ref[...], 0.0)


_degp_spec = pl.BlockSpec((NC, RB, 16), lambda i: (0, i, 0))
_row_spec = pl.BlockSpec((RB, D), lambda i: (i, 0))
_agg_spec = pl.BlockSpec((NC, RB, D), lambda i: (0, i, 0))
_w_spec = pl.BlockSpec((D, D), lambda i: (0, 0))
_b_spec = pl.BlockSpec((1, D), lambda i: (0, 0))

_tc_first = pl.pallas_call(
    _tc_first_body,
    grid=(GRID,),
    in_specs=[_degp_spec, _row_spec, _w_spec],
    out_specs=_row_spec,
    out_shape=jax.ShapeDtypeStruct((N_NODES, D), jnp.float32),
)

_tc_mid = pl.pallas_call(
    _tc_mid_body,
    grid=(GRID,),
    in_specs=[_degp_spec, _agg_spec, _row_spec, _b_spec, _w_spec],
    out_specs=_row_spec,
    out_shape=jax.ShapeDtypeStruct((N_NODES, D), jnp.float32),
)

_tc_last = pl.pallas_call(
    _tc_last_body,
    grid=(GRID,),
    in_specs=[_degp_spec, _agg_spec, _row_spec, _b_spec],
    out_specs=_row_spec,
    out_shape=jax.ShapeDtypeStruct((N_NODES, D), jnp.float32),
)


def kernel(x, edge_index, W1, b1, W2, b2):
    src = edge_index[0].astype(jnp.int32).reshape(NW, NB, K)
    dst = edge_index[1].astype(jnp.int32).reshape(NW, NB, K)
    dst_d = edge_index[1].astype(jnp.int32).reshape(NW, NBD, KD)
    ones16 = jnp.ones((KD, 16), jnp.float32)
    zeros16 = jnp.zeros((RPT, 16), jnp.float32)
    zerosD = jnp.zeros((RPT, D), jnp.float32)
    b1r = b1.reshape(1, D)
    b2r = b2.reshape(1, D)

    sc_degree, sc_aggregate = _sc_kernels()
    degp = sc_degree(dst_d, ones16, zeros16)
    y1 = _tc_first(degp, x, W1)
    agg1 = sc_aggregate(y1, src, dst, zerosD)
    y2 = _tc_mid(degp, agg1, y1, b1r, W2)
    agg2 = sc_aggregate(y2, src, dst, zerosD)
    out = _tc_last(degp, agg2, y2, b2r)
    return out
